# Initial kernel scaffold; baseline (speedup 1.0000x reference)
import jax
import jax.numpy as jnp
from jax import lax
from jax.experimental import pallas as pl
from jax.experimental.pallas import tpu as pltpu
from jax.experimental.pallas import tpu_sc as plsc

N, C, H, W = 4, 96, 384, 384
HG, WG = 384, 384
P = HG * WG
NP = N * P
NWORK = 32
PTS_PER_W = NP // NWORK
CHUNK = 128
CHUNKS_PER_W = PTS_PER_W // CHUNK
L = 16
NV = C // L


def _grid_sample_body(xt, gx, gy, out,
                      gxv, gyv,
                      i00, i01, i10, i11,
                      w00, w01, w10, w11,
                      r00, r01, r10, r11,
                      outv, sem):
    wid = lax.axis_index("s") * 2 + lax.axis_index("c")
    base0 = wid * PTS_PER_W
    rowoff = lax.shift_right_logical(wid, 3) * (H * W)

    def chunk_body(ci, carry):
        base = base0 + ci * CHUNK
        pltpu.sync_copy(gx.at[pl.ds(base, CHUNK)], gxv)
        pltpu.sync_copy(gy.at[pl.ds(base, CHUNK)], gyv)

        for g in range(CHUNK // L):
            sl = pl.ds(g * L, L)
            gxg = gxv[sl]
            gyg = gyv[sl]
            ix = gxg * (W * 0.5) + (W - 1.0) * 0.5
            iy = gyg * (H * 0.5) + (H - 1.0) * 0.5
            ix0t = ix.astype(jnp.int32)
            ix0 = jnp.where(ix0t.astype(jnp.float32) > ix, ix0t - 1, ix0t)
            iy0t = iy.astype(jnp.int32)
            iy0 = jnp.where(iy0t.astype(jnp.float32) > iy, iy0t - 1, iy0t)
            fx = ix - ix0.astype(jnp.float32)
            fy = iy - iy0.astype(jnp.float32)
            ix1 = ix0 + 1
            iy1 = iy0 + 1
            wx0 = jnp.where((ix0 >= 0) & (ix0 <= W - 1), 1.0 - fx, 0.0)
            wx1 = jnp.where((ix1 >= 0) & (ix1 <= W - 1), fx, 0.0)
            wy0 = jnp.where((iy0 >= 0) & (iy0 <= H - 1), 1.0 - fy, 0.0)
            wy1 = jnp.where((iy1 >= 0) & (iy1 <= H - 1), fy, 0.0)
            ix0c = jnp.minimum(jnp.maximum(ix0, 0), W - 1)
            ix1c = jnp.minimum(jnp.maximum(ix1, 0), W - 1)
            iy0c = jnp.minimum(jnp.maximum(iy0, 0), H - 1)
            iy1c = jnp.minimum(jnp.maximum(iy1, 0), H - 1)
            r0 = rowoff + iy0c * W
            r1 = rowoff + iy1c * W
            i00[sl] = r0 + ix0c
            i01[sl] = r0 + ix1c
            i10[sl] = r1 + ix0c
            i11[sl] = r1 + ix1c
            w00[sl] = wy0 * wx0
            w01[sl] = wy0 * wx1
            w10[sl] = wy1 * wx0
            w11[sl] = wy1 * wx1

        d0 = pltpu.async_copy(xt.at[i00], r00, sem)
        d1 = pltpu.async_copy(xt.at[i01], r01, sem)
        d2 = pltpu.async_copy(xt.at[i10], r10, sem)
        d3 = pltpu.async_copy(xt.at[i11], r11, sem)
        d0.wait()
        d1.wait()
        d2.wait()
        d3.wait()

        def pbody(p, c2):
            pv = jnp.full((L,), p, jnp.int32)
            s00 = plsc.load_gather(w00, [pv])
            s01 = plsc.load_gather(w01, [pv])
            s10 = plsc.load_gather(w10, [pv])
            s11 = plsc.load_gather(w11, [pv])
            for j in range(NV):
                cs = pl.ds(j * L, L)
                acc = (r00[p, cs] * s00 + r01[p, cs] * s01
                       + r10[p, cs] * s10 + r11[p, cs] * s11)
                outv[p, cs] = acc
            return c2

        lax.fori_loop(0, CHUNK, pbody, 0)

        pltpu.sync_copy(outv, out.at[pl.ds(base, CHUNK)])
        return carry

    lax.fori_loop(0, CHUNKS_PER_W, chunk_body, 0)


@jax.jit
def kernel(x, grid):
    xt = jnp.transpose(x, (0, 2, 3, 1)).reshape(N * H * W, C)
    gx = grid[..., 0].reshape(NP)
    gy = grid[..., 1].reshape(NP)

    mesh = plsc.VectorSubcoreMesh(core_axis_name="c", subcore_axis_name="s")
    run = pl.kernel(
        _grid_sample_body,
        mesh=mesh,
        out_type=jax.ShapeDtypeStruct((NP, C), jnp.float32),
        scratch_types=[
            pltpu.VMEM((CHUNK,), jnp.float32),
            pltpu.VMEM((CHUNK,), jnp.float32),
            pltpu.VMEM((CHUNK,), jnp.int32),
            pltpu.VMEM((CHUNK,), jnp.int32),
            pltpu.VMEM((CHUNK,), jnp.int32),
            pltpu.VMEM((CHUNK,), jnp.int32),
            pltpu.VMEM((CHUNK,), jnp.float32),
            pltpu.VMEM((CHUNK,), jnp.float32),
            pltpu.VMEM((CHUNK,), jnp.float32),
            pltpu.VMEM((CHUNK,), jnp.float32),
            pltpu.VMEM((CHUNK, C), jnp.float32),
            pltpu.VMEM((CHUNK, C), jnp.float32),
            pltpu.VMEM((CHUNK, C), jnp.float32),
            pltpu.VMEM((CHUNK, C), jnp.float32),
            pltpu.VMEM((CHUNK, C), jnp.float32),
            pltpu.SemaphoreType.DMA,
        ],
    )
    yt = run(xt, gx, gy)
    return yt.reshape(N, HG, WG, C).transpose(0, 3, 1, 2)


# speedup vs baseline: 1.5476x; 1.5476x over previous
import jax
import jax.numpy as jnp
from jax import lax
from jax.experimental import pallas as pl
from jax.experimental.pallas import tpu as pltpu
from jax.experimental.pallas import tpu_sc as plsc

N, C, H, W = 4, 96, 384, 384
HG, WG = 384, 384
P = HG * WG
NP = N * P
NWORK = 32
PTS_PER_W = NP // NWORK
CHUNK = 128
CHUNKS_PER_W = PTS_PER_W // CHUNK
L = 16
NV = C // L
CP = 128


def _grid_sample_body(xt, gx, gy, out,
                      gxv, gyv,
                      i00, i01, i10, i11,
                      w00, w01, w10, w11,
                      r00, r01, r10, r11,
                      outv, sem):
    wid = lax.axis_index("s") * 2 + lax.axis_index("c")
    base0 = wid * PTS_PER_W
    rowoff = lax.shift_right_logical(wid, 3) * (H * W)

    def chunk_body(ci, carry):
        base = base0 + ci * CHUNK
        pltpu.sync_copy(gx.at[pl.ds(base, CHUNK)], gxv)
        pltpu.sync_copy(gy.at[pl.ds(base, CHUNK)], gyv)

        for g in range(CHUNK // L):
            sl = pl.ds(g * L, L)
            gxg = gxv[sl]
            gyg = gyv[sl]
            ix = gxg * (W * 0.5) + (W - 1.0) * 0.5
            iy = gyg * (H * 0.5) + (H - 1.0) * 0.5
            ix0t = ix.astype(jnp.int32)
            ix0 = jnp.where(ix0t.astype(jnp.float32) > ix, ix0t - 1, ix0t)
            iy0t = iy.astype(jnp.int32)
            iy0 = jnp.where(iy0t.astype(jnp.float32) > iy, iy0t - 1, iy0t)
            fx = ix - ix0.astype(jnp.float32)
            fy = iy - iy0.astype(jnp.float32)
            ix1 = ix0 + 1
            iy1 = iy0 + 1
            wx0 = jnp.where((ix0 >= 0) & (ix0 <= W - 1), 1.0 - fx, 0.0)
            wx1 = jnp.where((ix1 >= 0) & (ix1 <= W - 1), fx, 0.0)
            wy0 = jnp.where((iy0 >= 0) & (iy0 <= H - 1), 1.0 - fy, 0.0)
            wy1 = jnp.where((iy1 >= 0) & (iy1 <= H - 1), fy, 0.0)
            ix0c = jnp.minimum(jnp.maximum(ix0, 0), W - 1)
            ix1c = jnp.minimum(jnp.maximum(ix1, 0), W - 1)
            iy0c = jnp.minimum(jnp.maximum(iy0, 0), H - 1)
            iy1c = jnp.minimum(jnp.maximum(iy1, 0), H - 1)
            r0 = rowoff + iy0c * W
            r1 = rowoff + iy1c * W
            i00[sl] = r0 + ix0c
            i01[sl] = r0 + ix1c
            i10[sl] = r1 + ix0c
            i11[sl] = r1 + ix1c
            w00[sl] = wy0 * wx0
            w01[sl] = wy0 * wx1
            w10[sl] = wy1 * wx0
            w11[sl] = wy1 * wx1

        d0 = pltpu.async_copy(xt.at[i00], r00, sem)
        d1 = pltpu.async_copy(xt.at[i01], r01, sem)
        d2 = pltpu.async_copy(xt.at[i10], r10, sem)
        d3 = pltpu.async_copy(xt.at[i11], r11, sem)
        d0.wait()
        d1.wait()
        d2.wait()
        d3.wait()

        def gbody(g, c2):
            go = g * L
            w00g = w00[pl.ds(go, L)]
            w01g = w01[pl.ds(go, L)]
            w10g = w10[pl.ds(go, L)]
            w11g = w11[pl.ds(go, L)]
            for k in range(L):
                p = go + k
                s00 = jnp.full((L,), w00g[k])
                s01 = jnp.full((L,), w01g[k])
                s10 = jnp.full((L,), w10g[k])
                s11 = jnp.full((L,), w11g[k])
                for j in range(NV):
                    cs = pl.ds(j * L, L)
                    acc = (r00[p, cs] * s00 + r01[p, cs] * s01
                           + r10[p, cs] * s10 + r11[p, cs] * s11)
                    outv[p, cs] = acc
            return c2

        lax.fori_loop(0, CHUNK // L, gbody, 0)

        pltpu.sync_copy(outv, out.at[pl.ds(base, CHUNK)])
        return carry

    lax.fori_loop(0, CHUNKS_PER_W, chunk_body, 0)


@jax.jit
def kernel(x, grid):
    xt = jnp.pad(jnp.transpose(x, (0, 2, 3, 1)).reshape(N * H * W, C),
                 ((0, 0), (0, CP - C)))
    gx = grid[..., 0].reshape(NP)
    gy = grid[..., 1].reshape(NP)

    mesh = plsc.VectorSubcoreMesh(core_axis_name="c", subcore_axis_name="s")
    run = pl.kernel(
        _grid_sample_body,
        mesh=mesh,
        out_type=jax.ShapeDtypeStruct((NP, C), jnp.float32),
        scratch_types=[
            pltpu.VMEM((CHUNK,), jnp.float32),
            pltpu.VMEM((CHUNK,), jnp.float32),
            pltpu.VMEM((CHUNK,), jnp.int32),
            pltpu.VMEM((CHUNK,), jnp.int32),
            pltpu.VMEM((CHUNK,), jnp.int32),
            pltpu.VMEM((CHUNK,), jnp.int32),
            pltpu.VMEM((CHUNK,), jnp.float32),
            pltpu.VMEM((CHUNK,), jnp.float32),
            pltpu.VMEM((CHUNK,), jnp.float32),
            pltpu.VMEM((CHUNK,), jnp.float32),
            pltpu.VMEM((CHUNK, CP), jnp.float32),
            pltpu.VMEM((CHUNK, CP), jnp.float32),
            pltpu.VMEM((CHUNK, CP), jnp.float32),
            pltpu.VMEM((CHUNK, CP), jnp.float32),
            pltpu.VMEM((CHUNK, C), jnp.float32),
            pltpu.SemaphoreType.DMA,
        ],
    )
    yt = run(xt, gx, gy)
    return yt.reshape(N, HG, WG, C).transpose(0, 3, 1, 2)


# speedup vs baseline: 2.2458x; 1.4511x over previous
import jax
import jax.numpy as jnp
from jax import lax
from jax.experimental import pallas as pl
from jax.experimental.pallas import tpu as pltpu
from jax.experimental.pallas import tpu_sc as plsc

N, C, H, W = 4, 96, 384, 384
HG, WG = 384, 384
P = HG * WG
NP = N * P
NWORK = 32
PTS_PER_W = NP // NWORK
CHUNK = 64
NCH = PTS_PER_W // CHUNK
L = 16
GPC = CHUNK // L
NV = C // L
CP = 128


def _sc_body(xt, gx, gy, out,
             gxv, gyv,
             i00, i01, i10, i11,
             w00, w01, w10, w11,
             r00, r01, r10, r11,
             outv,
             gsem0, gsem1, rsem0, rsem1, osem0, osem1):
    wid = lax.axis_index("s") * 2 + lax.axis_index("c")
    base0 = wid * PTS_PER_W
    rowoff = lax.shift_right_logical(wid, 3) * (H * W)
    gsems = (gsem0, gsem1)
    rsems = (rsem0, rsem1)
    osems = (osem0, osem1)

    def idxw(s):
        for g in range(GPC):
            sl = pl.ds(g * L, L)
            gxg = gxv[s, sl]
            gyg = gyv[s, sl]
            ix = gxg * (W * 0.5) + (W - 1.0) * 0.5
            iy = gyg * (H * 0.5) + (H - 1.0) * 0.5
            ix0t = ix.astype(jnp.int32)
            ix0 = jnp.where(ix0t.astype(jnp.float32) > ix, ix0t - 1, ix0t)
            iy0t = iy.astype(jnp.int32)
            iy0 = jnp.where(iy0t.astype(jnp.float32) > iy, iy0t - 1, iy0t)
            fx = ix - ix0.astype(jnp.float32)
            fy = iy - iy0.astype(jnp.float32)
            ix1 = ix0 + 1
            iy1 = iy0 + 1
            wx0 = jnp.where((ix0 >= 0) & (ix0 <= W - 1), 1.0 - fx, 0.0)
            wx1 = jnp.where((ix1 >= 0) & (ix1 <= W - 1), fx, 0.0)
            wy0 = jnp.where((iy0 >= 0) & (iy0 <= H - 1), 1.0 - fy, 0.0)
            wy1 = jnp.where((iy1 >= 0) & (iy1 <= H - 1), fy, 0.0)
            ix0c = jnp.minimum(jnp.maximum(ix0, 0), W - 1)
            ix1c = jnp.minimum(jnp.maximum(ix1, 0), W - 1)
            iy0c = jnp.minimum(jnp.maximum(iy0, 0), H - 1)
            iy1c = jnp.minimum(jnp.maximum(iy1, 0), H - 1)
            r0 = rowoff + iy0c * W
            r1 = rowoff + iy1c * W
            i00[s, sl] = r0 + ix0c
            i01[s, sl] = r0 + ix1c
            i10[s, sl] = r1 + ix0c
            i11[s, sl] = r1 + ix1c
            w00[s, sl] = wy0 * wx0
            w01[s, sl] = wy0 * wx1
            w10[s, sl] = wy1 * wx0
            w11[s, sl] = wy1 * wx1

    def gather_copies(s):
        sem = rsems[s]
        return (
            pltpu.make_async_copy(xt.at[i00.at[s]], r00.at[s], sem),
            pltpu.make_async_copy(xt.at[i01.at[s]], r01.at[s], sem),
            pltpu.make_async_copy(xt.at[i10.at[s]], r10.at[s], sem),
            pltpu.make_async_copy(xt.at[i11.at[s]], r11.at[s], sem),
        )

    def grid_copies(s, base):
        sem = gsems[s]
        return (
            pltpu.make_async_copy(gx.at[pl.ds(base, CHUNK)], gxv.at[s], sem),
            pltpu.make_async_copy(gy.at[pl.ds(base, CHUNK)], gyv.at[s], sem),
        )

    def store_copy(s, base):
        return pltpu.make_async_copy(
            outv.at[s], out.at[pl.ds(base, CHUNK)], osems[s])

    def blend(s):
        def gbody(g, c2):
            go = g * L
            w00g = w00[s, pl.ds(go, L)]
            w01g = w01[s, pl.ds(go, L)]
            w10g = w10[s, pl.ds(go, L)]
            w11g = w11[s, pl.ds(go, L)]
            for k in range(L):
                p = go + k
                s00 = jnp.full((L,), w00g[k])
                s01 = jnp.full((L,), w01g[k])
                s10 = jnp.full((L,), w10g[k])
                s11 = jnp.full((L,), w11g[k])
                for j in range(NV):
                    cs = pl.ds(j * L, L)
                    acc = (r00[s, p, cs] * s00 + r01[s, p, cs] * s01
                           + r10[s, p, cs] * s10 + r11[s, p, cs] * s11)
                    outv[s, p, cs] = acc
            return c2

        lax.fori_loop(0, GPC, gbody, 0)

    def proc(i, s):
        q = 1 - s

        @pl.when(i + 1 < NCH)
        def _():
            for cp in grid_copies(q, 0):
                cp.wait()
            idxw(q)
            for cp in gather_copies(q):
                cp.start()

        @pl.when(i + 2 < NCH)
        def _():
            for cp in grid_copies(s, base0 + (i + 2) * CHUNK):
                cp.start()

        for cp in gather_copies(s):
            cp.wait()

        @pl.when(i >= 2)
        def _():
            store_copy(s, 0).wait()

        blend(s)
        store_copy(s, base0 + i * CHUNK).start()

    for cp in grid_copies(0, base0):
        cp.start()
    for cp in grid_copies(0, base0):
        cp.wait()
    idxw(0)
    for cp in gather_copies(0):
        cp.start()
    for cp in grid_copies(1, base0 + CHUNK):
        cp.start()

    def pair(sidx, c2):
        proc(sidx * 2, 0)
        proc(sidx * 2 + 1, 1)
        return c2

    lax.fori_loop(0, NCH // 2, pair, 0)

    store_copy(0, 0).wait()
    store_copy(1, 0).wait()


@jax.jit
def kernel(x, grid):
    xt = jnp.pad(jnp.transpose(x, (0, 2, 3, 1)).reshape(N * H * W, C),
                 ((0, 0), (0, CP - C)))
    gx = grid[..., 0].reshape(NP)
    gy = grid[..., 1].reshape(NP)

    mesh = plsc.VectorSubcoreMesh(core_axis_name="c", subcore_axis_name="s")
    run = pl.kernel(
        _sc_body,
        mesh=mesh,
        out_type=jax.ShapeDtypeStruct((NP, C), jnp.float32),
        scratch_types=[
            pltpu.VMEM((2, CHUNK), jnp.float32),
            pltpu.VMEM((2, CHUNK), jnp.float32),
            pltpu.VMEM((2, CHUNK), jnp.int32),
            pltpu.VMEM((2, CHUNK), jnp.int32),
            pltpu.VMEM((2, CHUNK), jnp.int32),
            pltpu.VMEM((2, CHUNK), jnp.int32),
            pltpu.VMEM((2, CHUNK), jnp.float32),
            pltpu.VMEM((2, CHUNK), jnp.float32),
            pltpu.VMEM((2, CHUNK), jnp.float32),
            pltpu.VMEM((2, CHUNK), jnp.float32),
            pltpu.VMEM((2, CHUNK, CP), jnp.float32),
            pltpu.VMEM((2, CHUNK, CP), jnp.float32),
            pltpu.VMEM((2, CHUNK, CP), jnp.float32),
            pltpu.VMEM((2, CHUNK, CP), jnp.float32),
            pltpu.VMEM((2, CHUNK, C), jnp.float32),
            pltpu.SemaphoreType.DMA,
            pltpu.SemaphoreType.DMA,
            pltpu.SemaphoreType.DMA,
            pltpu.SemaphoreType.DMA,
            pltpu.SemaphoreType.DMA,
            pltpu.SemaphoreType.DMA,
        ],
    )
    yt = run(xt, gx, gy)
    return yt.reshape(N, HG, WG, C).transpose(0, 3, 1, 2)
